# baseline (speedup 1.0000x reference)
import functools

import jax
import jax.numpy as jnp
from jax import lax
from jax.experimental import pallas as pl
from jax.experimental.pallas import tpu as pltpu
from jax.experimental.pallas import tpu_sc as plsc

_NC = 2
_NS = 16
_NW = _NC * _NS

_IB = 128
_TS = 16
_L = 16


def _sc_hilbert_flatten(x4, idxs2, B, S, C):
    S2 = S * S
    BS2 = B * S2
    RW = BS2 // _NW
    NIR = RW // _IB
    WPB = _NW // B
    TR = _TS * _TS
    HR = TR // 2
    NTW = RW // TR
    NITER = NTW // 2
    LS = S.bit_length() - 1
    LTS = _TS.bit_length() - 1
    assert (1 << LS) == S and RW % TR == 0 and TR % _IB == 0

    mesh = plsc.VectorSubcoreMesh(core_axis_name="c", subcore_axis_name="s")

    @functools.partial(
        pl.kernel,
        mesh=mesh,
        out_type=jax.ShapeDtypeStruct((B, S2, C), jnp.float32),
        scratch_types=[
            pltpu.VMEM((NIR, _IB), jnp.int32),
            pltpu.VMEM((2 * TR, C), jnp.float32),
            pltpu.VMEM((2, HR, C), jnp.float32),
            [pltpu.SemaphoreType.DMA] * 2,
            [pltpu.SemaphoreType.DMA] * 2,
        ],
    )
    def k(x_hbm, idxs_hbm, out_hbm, idx_v, tbuf, obuf, lsems, wsems):
        cid = lax.axis_index("c")
        sid = lax.axis_index("s")
        wid = sid * _NC + cid
        b = wid // WPB
        obase0 = (wid % WPB) * RW

        pltpu.sync_copy(idxs_hbm.at[pl.ds((wid % WPB) * NIR, NIR)], idx_v)

        def fire_loads(t, tb):
            g0 = idx_v[(TR // _IB) * t, pl.ds(0, _L)][0]
            y0 = ((g0 >> LS) >> LTS) << LTS
            x0 = pl.multiple_of(((g0 & (S - 1)) >> LTS) << LTS, _TS)
            for yy in range(_TS):
                pltpu.async_copy(
                    x_hbm.at[b, y0 + yy, pl.ds(x0, _TS)],
                    tbuf.at[pl.ds(tb * TR + yy * _TS, _TS)],
                    lsems[tb],
                )

        def wait_loads(tb):
            for yy in range(_TS):
                pltpu.make_async_copy(
                    out_hbm.at[0, pl.ds(0, _TS)],
                    tbuf.at[pl.ds(tb * TR + yy * _TS, _TS)],
                    lsems[tb],
                ).wait()

        def fire_write(t, half, ob):
            pltpu.async_copy(
                obuf.at[ob],
                out_hbm.at[
                    b,
                    pl.ds(pl.multiple_of(obase0 + t * TR + half * HR, HR), HR),
                ],
                wsems[ob],
            )

        def wait_write(ob):
            pltpu.make_async_copy(
                obuf.at[ob], out_hbm.at[0, pl.ds(0, HR)], wsems[ob]
            ).wait()

        def permute_half(t, tb, half, ob):
            irow = (TR // _IB) * t + half

            def grp_body(grp, carry):
                g = idx_v[irow, pl.ds(grp * _L, _L)]
                pvec = (((g >> LS) & (_TS - 1)) << LTS) | (g & (_TS - 1))
                pvec = pvec + tb * TR
                for u in range(_L):
                    p = pvec[u]
                    for v in range(C // _L):
                        sl = pl.ds(v * _L, _L)
                        obuf[ob, grp * _L + u, sl] = tbuf[p, sl]
                return carry

            lax.fori_loop(0, HR // _L, grp_body, 0)

        def process(i, t, tb):
            wait_loads(tb)

            for half in range(2):
                ob = half

                @pl.when(i >= 1)
                def _():
                    wait_write(ob)

                permute_half(t, tb, half, ob)
                fire_write(t, half, ob)

            @pl.when(i < NITER - 1)
            def _():
                fire_loads(t + 2, tb)

        fire_loads(0, 0)
        fire_loads(1, 1)

        def body(i, carry):
            process(i, 2 * i, 0)
            process(i, 2 * i + 1, 1)
            return carry

        lax.fori_loop(0, NITER, body, 0)
        wait_write(0)
        wait_write(1)

    return k(x4, idxs2)


def kernel(inputs, idxs):
    B, S, _, C = inputs.shape
    S2 = S * S
    idxs2 = idxs.reshape(S2 // _IB, _IB)
    return _sc_hilbert_flatten(inputs, idxs2, B, S, C)


# speedup vs baseline: 1.0898x; 1.0898x over previous
import functools

import jax
import jax.numpy as jnp
from jax import lax
from jax.experimental import pallas as pl
from jax.experimental.pallas import tpu as pltpu
from jax.experimental.pallas import tpu_sc as plsc

_NC = 2
_NS = 16
_NW = _NC * _NS

_IB = 128
_TS = 16
_L = 16


def _sc_hilbert_flatten(x2, idxs2, B, S, C):
    S2 = S * S
    BS2 = B * S2
    RW = BS2 // _NW
    NIR = RW // _IB
    WPB = _NW // B
    TR = _TS * _TS
    HR = TR // 2
    NTW = RW // TR
    NITER = NTW // 2
    LS = S.bit_length() - 1
    LTS = _TS.bit_length() - 1
    assert (1 << LS) == S and RW % TR == 0 and TR % _IB == 0

    mesh = plsc.VectorSubcoreMesh(core_axis_name="c", subcore_axis_name="s")

    @functools.partial(
        pl.kernel,
        mesh=mesh,
        out_type=jax.ShapeDtypeStruct((BS2, C), jnp.float32),
        scratch_types=[
            pltpu.VMEM((NIR, _IB), jnp.int32),
            pltpu.VMEM((2, _TS, _TS, C), jnp.float32),
            pltpu.VMEM((2, HR, C), jnp.float32),
            [pltpu.SemaphoreType.DMA] * 2,
            [pltpu.SemaphoreType.DMA] * 2,
        ],
    )
    def k(x_hbm, idxs_hbm, out_hbm, idx_v, tbuf, obuf, lsems, wsems):
        cid = lax.axis_index("c")
        sid = lax.axis_index("s")
        wid = sid * _NC + cid
        wbase = wid * RW
        b = wid // WPB

        pltpu.sync_copy(idxs_hbm.at[pl.ds((wid % WPB) * NIR, NIR)], idx_v)

        def fire_loads(t, tb):
            g0 = idx_v[(TR // _IB) * t, pl.ds(0, 16)][0]
            y0 = ((g0 >> LS) >> LTS) << LTS
            x0 = ((g0 & (S - 1)) >> LTS) << LTS
            rowbase = pl.multiple_of(b * S2 + y0 * S + x0, _TS)
            for yy in range(_TS):
                pltpu.async_copy(
                    x_hbm.at[pl.ds(pl.multiple_of(rowbase + yy * S, _TS), _TS)],
                    tbuf.at[tb, yy],
                    lsems[tb],
                )

        def wait_loads(tb):
            for yy in range(_TS):
                pltpu.make_async_copy(
                    out_hbm.at[pl.ds(0, _TS)], tbuf.at[tb, yy], lsems[tb]
                ).wait()

        def fire_write(t, half, ob):
            pltpu.async_copy(
                obuf.at[ob],
                out_hbm.at[
                    pl.ds(pl.multiple_of(wbase + t * TR + half * HR, HR), HR)
                ],
                wsems[ob],
            )

        def wait_write(ob):
            pltpu.make_async_copy(
                obuf.at[ob], out_hbm.at[pl.ds(0, HR)], wsems[ob]
            ).wait()

        def permute_half(t, tb, half, ob):
            irow = (TR // _IB) * t + half

            def grp_body(grp, carry):
                g = idx_v[irow, pl.ds(grp * _L, _L)]
                for u in range(_L):
                    gu = g[u]
                    ph = (gu >> LS) & (_TS - 1)
                    pw = gu & (_TS - 1)
                    for v in range(C // _L):
                        sl = pl.ds(v * _L, _L)
                        obuf[ob, grp * _L + u, sl] = tbuf[tb, ph, pw, sl]
                return carry

            lax.fori_loop(0, HR // _L, grp_body, 0)

        def process(i, t, tb):
            wait_loads(tb)

            for half in range(2):
                ob = half

                @pl.when(i >= 1)
                def _():
                    wait_write(ob)

                permute_half(t, tb, half, ob)
                fire_write(t, half, ob)

            @pl.when(i < NITER - 1)
            def _():
                fire_loads(t + 2, tb)

        fire_loads(0, 0)
        fire_loads(1, 1)

        def body(i, carry):
            process(i, 2 * i, 0)
            process(i, 2 * i + 1, 1)
            return carry

        lax.fori_loop(0, NITER, body, 0)
        wait_write(0)
        wait_write(1)

    return k(x2, idxs2)


def kernel(inputs, idxs):
    B, S, _, C = inputs.shape
    S2 = S * S
    x2 = inputs.reshape(B * S2, C)
    idxs2 = idxs.reshape(S2 // _IB, _IB)
    out = _sc_hilbert_flatten(x2, idxs2, B, S, C)
    return out.reshape(B, S2, C)
